# baseline (speedup 1.0000x reference)
import jax
import jax.numpy as jnp
from jax import lax
from jax.experimental import pallas as pl
from jax.experimental.pallas import tpu as pltpu
from jax.experimental.pallas import tpu_sc as plsc

N_NODES = 10000
N_EDGES = 320000
D = 128
NC = 2
NS = 16
E_PER_TILE = N_EDGES // (NC * NS)
CHUNK = 40
N_CHUNKS = E_PER_TILE // CHUNK
N_PAD = 10240
ROWS_PER_TILE = N_PAD // NS
ZROWS = 128


def _sc_agg_body(feat_hbm, src_hbm, dst_hbm, out0_hbm, out1_hbm,
                 src_v, dst_v, rows_a, rows_b, rows_c, acc,
                 ga, gb, gc, sa, sb, sc):
    c = lax.axis_index("c")
    s = lax.axis_index("s")
    wid = c * NS + s
    ebase = pl.multiple_of(wid * E_PER_TILE, 8)

    pltpu.sync_copy(src_hbm.at[pl.ds(ebase, E_PER_TILE)], src_v)
    pltpu.sync_copy(dst_hbm.at[pl.ds(ebase, E_PER_TILE)], dst_v)

    def _gath(n, buf, sem):
        off = pl.multiple_of(n * CHUNK, 8)
        return pltpu.make_async_copy(
            feat_hbm.at[src_v.at[pl.ds(off, CHUNK)]], buf, sem)

    def _sdesc(n, buf, sem):
        off = pl.multiple_of(n * CHUNK, 8)
        return pltpu.make_async_copy(
            buf, acc.at[dst_v.at[pl.ds(off, CHUNK)]], sem)

    _gath(0, rows_a, ga).start()
    _gath(1, rows_b, gb).start()
    zeros16 = jnp.zeros((16,), jnp.float32)

    def _zfill(r, carry):
        for c8 in range(D // 16):
            rows_c[r, pl.ds(c8 * 16, 16)] = zeros16
        return carry

    lax.fori_loop(0, CHUNK, _zfill, 0)
    for k in range(ROWS_PER_TILE // CHUNK):
        off = pl.multiple_of(s * ROWS_PER_TILE + k * CHUNK, 8)
        pltpu.sync_copy(rows_c, acc.at[pl.ds(off, CHUNK)])
    plsc.subcore_barrier()

    gwait = lambda n, buf, sem: _gath(n, buf, sem).wait()
    sstart = lambda n, buf, sem: _sdesc(n, buf, sem).start(add=True)
    swait = lambda n, buf, sem: _sdesc(n, buf, sem).wait()

    gwait(0, rows_a, ga); sstart(0, rows_a, sa); _gath(2, rows_c, gc).start()
    gwait(1, rows_b, gb); sstart(1, rows_b, sb)
    swait(0, rows_a, sa); _gath(3, rows_a, ga).start()
    gwait(2, rows_c, gc); sstart(2, rows_c, sc)
    swait(1, rows_b, sb); _gath(4, rows_b, gb).start()

    def _body(g, carry):
        m = 3 * g + 3
        gwait(m, rows_a, ga); sstart(m, rows_a, sa)
        swait(m - 1, rows_c, sc); _gath(m + 2, rows_c, gc).start()
        gwait(m + 1, rows_b, gb); sstart(m + 1, rows_b, sb)
        swait(m, rows_a, sa); _gath(m + 3, rows_a, ga).start()
        gwait(m + 2, rows_c, gc); sstart(m + 2, rows_c, sc)
        swait(m + 1, rows_b, sb); _gath(m + 4, rows_b, gb).start()
        return carry

    lax.fori_loop(0, (N_CHUNKS - 7) // 3, _body, 0)

    gwait(246, rows_a, ga); sstart(246, rows_a, sa)
    swait(245, rows_c, sc); _gath(248, rows_c, gc).start()
    gwait(247, rows_b, gb); sstart(247, rows_b, sb)
    swait(246, rows_a, sa); _gath(249, rows_a, ga).start()
    gwait(248, rows_c, gc); sstart(248, rows_c, sc)
    gwait(249, rows_a, ga); sstart(249, rows_a, sa)
    swait(247, rows_b, sb)
    swait(248, rows_c, sc)
    swait(249, rows_a, sa)

    plsc.subcore_barrier()

    @pl.when(c == 0)
    def _():
        pltpu.sync_copy(acc.at[pl.ds(s * ROWS_PER_TILE, ROWS_PER_TILE)],
                        out0_hbm.at[pl.ds(s * ROWS_PER_TILE, ROWS_PER_TILE)])

    @pl.when(c == 1)
    def _():
        pltpu.sync_copy(acc.at[pl.ds(s * ROWS_PER_TILE, ROWS_PER_TILE)],
                        out1_hbm.at[pl.ds(s * ROWS_PER_TILE, ROWS_PER_TILE)])


def _sc_aggregate(features, src, dst):
    mesh = plsc.VectorSubcoreMesh(core_axis_name="c", subcore_axis_name="s")
    f32 = jnp.float32
    return pl.kernel(
        _sc_agg_body,
        mesh=mesh,
        out_type=[jax.ShapeDtypeStruct((N_PAD, D), f32),
                  jax.ShapeDtypeStruct((N_PAD, D), f32)],
        scratch_types=[
            pltpu.VMEM((E_PER_TILE,), jnp.int32),
            pltpu.VMEM((E_PER_TILE,), jnp.int32),
            pltpu.VMEM((CHUNK, D), f32),
            pltpu.VMEM((CHUNK, D), f32),
            pltpu.VMEM((CHUNK, D), f32),
            pltpu.VMEM_SHARED((N_PAD, D), f32),
            pltpu.SemaphoreType.DMA,
            pltpu.SemaphoreType.DMA,
            pltpu.SemaphoreType.DMA,
            pltpu.SemaphoreType.DMA,
            pltpu.SemaphoreType.DMA,
            pltpu.SemaphoreType.DMA,
        ],
    )(features, src, dst)


_BM = 2000


def _mm_body(p0_ref, p1_ref, w_ref, b_ref, o_ref):
    a = p0_ref[...] + p1_ref[...]
    o_ref[...] = jnp.dot(a, w_ref[...],
                         preferred_element_type=jnp.float32) + b_ref[...]


def _linear(p0, p1, W, b2d):
    return pl.pallas_call(
        _mm_body,
        grid=(N_NODES // _BM,),
        in_specs=[
            pl.BlockSpec((_BM, D), lambda i: (i, 0)),
            pl.BlockSpec((_BM, D), lambda i: (i, 0)),
            pl.BlockSpec((D, D), lambda i: (0, 0)),
            pl.BlockSpec((1, D), lambda i: (0, 0)),
        ],
        out_specs=pl.BlockSpec((_BM, D), lambda i: (i, 0)),
        out_shape=jax.ShapeDtypeStruct((N_NODES, D), jnp.float32),
    )(p0, p1, W, b2d)


def kernel(features, edge_index, W, b):
    src = edge_index[0].astype(jnp.int32)
    dst = edge_index[1].astype(jnp.int32)
    p0, p1 = _sc_aggregate(features, src, dst)
    return _linear(p0, p1, W, b.reshape(1, D))


# speedup vs baseline: 1.1802x; 1.1802x over previous
import jax
import jax.numpy as jnp
from jax import lax
from jax.experimental import pallas as pl
from jax.experimental.pallas import tpu as pltpu
from jax.experimental.pallas import tpu_sc as plsc

N_NODES = 10000
N_EDGES = 320000
D = 128
NC = 2
NS = 16
E_PER_TILE = N_EDGES // (NC * NS)
CHUNK = 64
N_FULL = E_PER_TILE // CHUNK
TAIL = E_PER_TILE - N_FULL * CHUNK
N_CHUNKS = N_FULL + 1
N_PAD = 10240
ROWS_PER_TILE = N_PAD // NS


def _sc_agg_body(feat_hbm, src_hbm, dst_hbm, out0_hbm, out1_hbm,
                 src_v, dst_v, rows_a, rows_b, rows_c, acc,
                 ga, gb, gc, sa, sb, sc):
    c = lax.axis_index("c")
    s = lax.axis_index("s")
    wid = c * NS + s
    ebase = pl.multiple_of(wid * E_PER_TILE, 8)

    pltpu.sync_copy(src_hbm.at[pl.ds(ebase, E_PER_TILE)], src_v)
    pltpu.sync_copy(dst_hbm.at[pl.ds(ebase, E_PER_TILE)], dst_v)

    bufs = (rows_a, rows_b, rows_c)
    gsems = (ga, gb, gc)
    ssems = (sa, sb, sc)

    def _sz(n):
        return TAIL if isinstance(n, int) and n == N_CHUNKS - 1 else CHUNK

    def _gath(n):
        j = n % 3 if isinstance(n, int) else None
        assert j is not None
        off = pl.multiple_of(n * CHUNK, 8)
        return pltpu.make_async_copy(
            feat_hbm.at[src_v.at[pl.ds(off, _sz(n))]],
            bufs[j].at[pl.ds(0, _sz(n))], gsems[j])

    def _gath_d(m, j):
        off = pl.multiple_of(m * CHUNK, 8)
        return pltpu.make_async_copy(
            feat_hbm.at[src_v.at[pl.ds(off, CHUNK)]], bufs[j], gsems[j])

    def _sdesc(n):
        j = n % 3
        off = pl.multiple_of(n * CHUNK, 8)
        return pltpu.make_async_copy(
            bufs[j].at[pl.ds(0, _sz(n))],
            acc.at[dst_v.at[pl.ds(off, _sz(n))]], ssems[j])

    def _sdesc_d(m, j):
        off = pl.multiple_of(m * CHUNK, 8)
        return pltpu.make_async_copy(
            bufs[j], acc.at[dst_v.at[pl.ds(off, CHUNK)]], ssems[j])

    _gath(0).start()
    _gath(1).start()
    zeros16 = jnp.zeros((16,), jnp.float32)

    def _zfill(r, carry):
        for c8 in range(D // 16):
            rows_c[r, pl.ds(c8 * 16, 16)] = zeros16
        return carry

    lax.fori_loop(0, CHUNK, _zfill, 0)
    for k in range(ROWS_PER_TILE // CHUNK):
        off = pl.multiple_of(s * ROWS_PER_TILE + k * CHUNK, 8)
        pltpu.sync_copy(rows_c, acc.at[pl.ds(off, CHUNK)])
    plsc.subcore_barrier()

    _gath(0).wait(); _sdesc(0).start(add=True); _gath(2).start()
    _gath(1).wait(); _sdesc(1).start(add=True)
    _sdesc(0).wait(); _gath(3).start()
    _gath(2).wait(); _sdesc(2).start(add=True)
    _sdesc(1).wait(); _gath(4).start()

    n_iter = (N_FULL - 4 - 3) // 3 + 1
    last_loop_m = 3 * (n_iter - 1) + 3 + 2

    def _body(g, carry):
        m = 3 * g + 3
        _gath_d(m, 0).wait(); _sdesc_d(m, 0).start(add=True)
        _sdesc_d(m - 1, 2).wait(); _gath_d(m + 2, 2).start()
        _gath_d(m + 1, 1).wait(); _sdesc_d(m + 1, 1).start(add=True)
        _sdesc_d(m, 0).wait(); _gath_d(m + 3, 0).start()
        _gath_d(m + 2, 2).wait(); _sdesc_d(m + 2, 2).start(add=True)
        _sdesc_d(m + 1, 1).wait(); _gath_d(m + 4, 1).start()
        return carry

    lax.fori_loop(0, n_iter, _body, 0)

    for m in range(last_loop_m + 1, N_CHUNKS):
        _gath(m).wait()
        _sdesc(m).start(add=True)
        if m + 2 < N_CHUNKS:
            _sdesc(m - 1).wait()
            _gath(m + 2).start()

    for m in range(N_CHUNKS - 3, N_CHUNKS):
        _sdesc(m).wait()

    plsc.subcore_barrier()

    @pl.when(c == 0)
    def _():
        pltpu.sync_copy(acc.at[pl.ds(s * ROWS_PER_TILE, ROWS_PER_TILE)],
                        out0_hbm.at[pl.ds(s * ROWS_PER_TILE, ROWS_PER_TILE)])

    @pl.when(c == 1)
    def _():
        pltpu.sync_copy(acc.at[pl.ds(s * ROWS_PER_TILE, ROWS_PER_TILE)],
                        out1_hbm.at[pl.ds(s * ROWS_PER_TILE, ROWS_PER_TILE)])


def _sc_aggregate(features, src, dst):
    mesh = plsc.VectorSubcoreMesh(core_axis_name="c", subcore_axis_name="s")
    f32 = jnp.float32
    return pl.kernel(
        _sc_agg_body,
        mesh=mesh,
        out_type=[jax.ShapeDtypeStruct((N_PAD, D), f32),
                  jax.ShapeDtypeStruct((N_PAD, D), f32)],
        scratch_types=[
            pltpu.VMEM((E_PER_TILE,), jnp.int32),
            pltpu.VMEM((E_PER_TILE,), jnp.int32),
            pltpu.VMEM((CHUNK, D), f32),
            pltpu.VMEM((CHUNK, D), f32),
            pltpu.VMEM((CHUNK, D), f32),
            pltpu.VMEM_SHARED((N_PAD, D), f32),
            pltpu.SemaphoreType.DMA,
            pltpu.SemaphoreType.DMA,
            pltpu.SemaphoreType.DMA,
            pltpu.SemaphoreType.DMA,
            pltpu.SemaphoreType.DMA,
            pltpu.SemaphoreType.DMA,
        ],
    )(features, src, dst)


_BM = 2000


def _mm_body(p0_ref, p1_ref, w_ref, b_ref, o_ref):
    a = p0_ref[...] + p1_ref[...]
    o_ref[...] = jnp.dot(a, w_ref[...],
                         preferred_element_type=jnp.float32) + b_ref[...]


def _linear(p0, p1, W, b2d):
    return pl.pallas_call(
        _mm_body,
        grid=(N_NODES // _BM,),
        in_specs=[
            pl.BlockSpec((_BM, D), lambda i: (i, 0)),
            pl.BlockSpec((_BM, D), lambda i: (i, 0)),
            pl.BlockSpec((D, D), lambda i: (0, 0)),
            pl.BlockSpec((1, D), lambda i: (0, 0)),
        ],
        out_specs=pl.BlockSpec((_BM, D), lambda i: (i, 0)),
        out_shape=jax.ShapeDtypeStruct((N_NODES, D), jnp.float32),
    )(p0, p1, W, b2d)


def kernel(features, edge_index, W, b):
    src = edge_index[0].astype(jnp.int32)
    dst = edge_index[1].astype(jnp.int32)
    p0, p1 = _sc_aggregate(features, src, dst)
    return _linear(p0, p1, W, b.reshape(1, D))


# speedup vs baseline: 1.2806x; 1.0851x over previous
import jax
import jax.numpy as jnp
from jax import lax
from jax.experimental import pallas as pl
from jax.experimental.pallas import tpu as pltpu
from jax.experimental.pallas import tpu_sc as plsc

N_NODES = 10000
N_EDGES = 320000
D = 128
NC = 2
NS = 16
E_PER_TILE = N_EDGES // (NC * NS)
CHUNK = 64
N_FULL = E_PER_TILE // CHUNK
TAIL = E_PER_TILE - N_FULL * CHUNK
N_CHUNKS = N_FULL + 1
N_PAD = 10240
ROWS_PER_TILE = N_PAD // NS


def _sc_agg_body(edges_hbm, feat_hbm, out0_hbm, out1_hbm,
                 src_v, dst_v, rows_a, rows_b, rows_c, acc,
                 ga, gb, gc, sa, sb, sc):
    c = lax.axis_index("c")
    s = lax.axis_index("s")
    wid = c * NS + s
    ebase = pl.multiple_of(wid * E_PER_TILE, 8)
    dbase = pl.multiple_of(N_EDGES + wid * E_PER_TILE, 8)

    cp_s = pltpu.make_async_copy(
        edges_hbm.at[pl.ds(ebase, E_PER_TILE)], src_v, ga)
    cp_d = pltpu.make_async_copy(
        edges_hbm.at[pl.ds(dbase, E_PER_TILE)], dst_v, sc)
    cp_s.start()
    cp_d.start()
    cp_s.wait()

    bufs = (rows_a, rows_b, rows_c)
    gsems = (ga, gb, gc)
    ssems = (sa, sb, sc)

    def _sz(n):
        return TAIL if isinstance(n, int) and n == N_CHUNKS - 1 else CHUNK

    def _gath(n):
        j = n % 3 if isinstance(n, int) else None
        assert j is not None
        off = pl.multiple_of(n * CHUNK, 8)
        return pltpu.make_async_copy(
            feat_hbm.at[src_v.at[pl.ds(off, _sz(n))]],
            bufs[j].at[pl.ds(0, _sz(n))], gsems[j])

    def _gath_d(m, j):
        off = pl.multiple_of(m * CHUNK, 8)
        return pltpu.make_async_copy(
            feat_hbm.at[src_v.at[pl.ds(off, CHUNK)]], bufs[j], gsems[j])

    def _sdesc(n):
        j = n % 3
        off = pl.multiple_of(n * CHUNK, 8)
        return pltpu.make_async_copy(
            bufs[j].at[pl.ds(0, _sz(n))],
            acc.at[dst_v.at[pl.ds(off, _sz(n))]], ssems[j])

    def _sdesc_d(m, j):
        off = pl.multiple_of(m * CHUNK, 8)
        return pltpu.make_async_copy(
            bufs[j], acc.at[dst_v.at[pl.ds(off, CHUNK)]], ssems[j])

    _gath(0).start()
    _gath(1).start()
    zeros16 = jnp.zeros((16,), jnp.float32)

    def _zfill(r, carry):
        for c8 in range(D // 16):
            rows_c[r, pl.ds(c8 * 16, 16)] = zeros16
        return carry

    lax.fori_loop(0, CHUNK, _zfill, 0)
    for k in range(ROWS_PER_TILE // CHUNK):
        off = pl.multiple_of(s * ROWS_PER_TILE + k * CHUNK, 8)
        pltpu.sync_copy(rows_c, acc.at[pl.ds(off, CHUNK)])
    cp_d.wait()
    plsc.subcore_barrier()

    _gath(0).wait(); _sdesc(0).start(add=True); _gath(2).start()
    _gath(1).wait(); _sdesc(1).start(add=True)
    _sdesc(0).wait(); _gath(3).start()
    _gath(2).wait(); _sdesc(2).start(add=True)
    _sdesc(1).wait(); _gath(4).start()

    n_iter = (N_FULL - 4 - 3) // 3 + 1
    last_loop_m = 3 * (n_iter - 1) + 3 + 2

    def _body(g, carry):
        m = 3 * g + 3
        _gath_d(m, 0).wait(); _sdesc_d(m, 0).start(add=True)
        _sdesc_d(m - 1, 2).wait(); _gath_d(m + 2, 2).start()
        _gath_d(m + 1, 1).wait(); _sdesc_d(m + 1, 1).start(add=True)
        _sdesc_d(m, 0).wait(); _gath_d(m + 3, 0).start()
        _gath_d(m + 2, 2).wait(); _sdesc_d(m + 2, 2).start(add=True)
        _sdesc_d(m + 1, 1).wait(); _gath_d(m + 4, 1).start()
        return carry

    lax.fori_loop(0, n_iter, _body, 0)

    for m in range(last_loop_m + 1, N_CHUNKS):
        _gath(m).wait()
        _sdesc(m).start(add=True)
        if m + 2 < N_CHUNKS:
            _sdesc(m - 1).wait()
            _gath(m + 2).start()

    for m in range(N_CHUNKS - 3, N_CHUNKS):
        _sdesc(m).wait()

    plsc.subcore_barrier()

    @pl.when(c == 0)
    def _():
        pltpu.sync_copy(acc.at[pl.ds(s * ROWS_PER_TILE, ROWS_PER_TILE)],
                        out0_hbm.at[pl.ds(s * ROWS_PER_TILE, ROWS_PER_TILE)])

    @pl.when(c == 1)
    def _():
        pltpu.sync_copy(acc.at[pl.ds(s * ROWS_PER_TILE, ROWS_PER_TILE)],
                        out1_hbm.at[pl.ds(s * ROWS_PER_TILE, ROWS_PER_TILE)])


def _sc_aggregate(edges, features):
    mesh = plsc.VectorSubcoreMesh(core_axis_name="c", subcore_axis_name="s")
    f32 = jnp.float32
    return pl.kernel(
        _sc_agg_body,
        mesh=mesh,
        out_type=[jax.ShapeDtypeStruct((N_PAD, D), f32),
                  jax.ShapeDtypeStruct((N_PAD, D), f32)],
        scratch_types=[
            pltpu.VMEM((E_PER_TILE,), jnp.int32),
            pltpu.VMEM((E_PER_TILE,), jnp.int32),
            pltpu.VMEM((CHUNK, D), f32),
            pltpu.VMEM((CHUNK, D), f32),
            pltpu.VMEM((CHUNK, D), f32),
            pltpu.VMEM_SHARED((N_PAD, D), f32),
            pltpu.SemaphoreType.DMA,
            pltpu.SemaphoreType.DMA,
            pltpu.SemaphoreType.DMA,
            pltpu.SemaphoreType.DMA,
            pltpu.SemaphoreType.DMA,
            pltpu.SemaphoreType.DMA,
        ],
    )(edges, features)


_BM = 2000


def _mm_body(p0_ref, p1_ref, w_ref, b_ref, o_ref):
    a = p0_ref[...] + p1_ref[...]
    o_ref[...] = jnp.dot(a, w_ref[...],
                         preferred_element_type=jnp.float32) + b_ref[...]


def _linear(p0, p1, W, b2d):
    return pl.pallas_call(
        _mm_body,
        grid=(N_NODES // _BM,),
        in_specs=[
            pl.BlockSpec((_BM, D), lambda i: (i, 0)),
            pl.BlockSpec((_BM, D), lambda i: (i, 0)),
            pl.BlockSpec((D, D), lambda i: (0, 0)),
            pl.BlockSpec((1, D), lambda i: (0, 0)),
        ],
        out_specs=pl.BlockSpec((_BM, D), lambda i: (i, 0)),
        out_shape=jax.ShapeDtypeStruct((N_NODES, D), jnp.float32),
    )(p0, p1, W, b2d)


def kernel(features, edge_index, W, b):
    edges = edge_index.astype(jnp.int32).reshape(-1)
    p0, p1 = _sc_aggregate(edges, features)
    return _linear(p0, p1, W, b.reshape(1, D))


# speedup vs baseline: 1.2981x; 1.0137x over previous
import jax
import jax.numpy as jnp
from jax import lax
from jax.experimental import pallas as pl
from jax.experimental.pallas import tpu as pltpu
from jax.experimental.pallas import tpu_sc as plsc

N_NODES = 10000
N_EDGES = 320000
D = 128
NC = 2
NS = 16
E_PER_TILE = N_EDGES // (NC * NS)
CHUNK = 64
N_FULL = E_PER_TILE // CHUNK
TAIL = E_PER_TILE - N_FULL * CHUNK
N_CHUNKS = N_FULL + 1
N_PAD = 10240
ROWS_PER_TILE = N_PAD // NS


def _sc_agg_body(edges_hbm, feat_hbm, out0_hbm, out1_hbm,
                 src_v, dst_v, rows_a, rows_b, rows_c, acc,
                 ga, gb, gc, sa, sb, sc):
    c = lax.axis_index("c")
    s = lax.axis_index("s")
    wid = c * NS + s
    ebase = pl.multiple_of(wid * E_PER_TILE, 8)
    dbase = pl.multiple_of(N_EDGES + wid * E_PER_TILE, 8)

    cp_s = pltpu.make_async_copy(
        edges_hbm.at[pl.ds(ebase, E_PER_TILE)], src_v, ga)
    cp_d = pltpu.make_async_copy(
        edges_hbm.at[pl.ds(dbase, E_PER_TILE)], dst_v, sc)
    cp_s.start()
    cp_d.start()
    cp_s.wait()

    bufs = (rows_a, rows_b, rows_c)
    gsems = (ga, gb, gc)
    ssems = (sa, sb, sc)

    def _sz(n):
        return TAIL if isinstance(n, int) and n == N_CHUNKS - 1 else CHUNK

    def _gath(n):
        j = n % 3 if isinstance(n, int) else None
        assert j is not None
        off = pl.multiple_of(n * CHUNK, 8)
        return pltpu.make_async_copy(
            feat_hbm.at[src_v.at[pl.ds(off, _sz(n))]],
            bufs[j].at[pl.ds(0, _sz(n))], gsems[j])

    def _gath_d(m, j):
        off = pl.multiple_of(m * CHUNK, 8)
        return pltpu.make_async_copy(
            feat_hbm.at[src_v.at[pl.ds(off, CHUNK)]], bufs[j], gsems[j])

    def _sdesc(n):
        j = n % 3
        off = pl.multiple_of(n * CHUNK, 8)
        return pltpu.make_async_copy(
            bufs[j].at[pl.ds(0, _sz(n))],
            acc.at[dst_v.at[pl.ds(off, _sz(n))]], ssems[j])

    def _sdesc_d(m, j):
        off = pl.multiple_of(m * CHUNK, 8)
        return pltpu.make_async_copy(
            bufs[j], acc.at[dst_v.at[pl.ds(off, CHUNK)]], ssems[j])

    _gath(0).start()
    _gath(1).start()
    zeros16 = jnp.zeros((16,), jnp.float32)

    def _zfill(r, carry):
        for c8 in range(D // 16):
            rows_c[r, pl.ds(c8 * 16, 16)] = zeros16
        return carry

    lax.fori_loop(0, CHUNK, _zfill, 0)
    for k in range(ROWS_PER_TILE // CHUNK):
        off = pl.multiple_of(s * ROWS_PER_TILE + k * CHUNK, 8)
        pltpu.sync_copy(rows_c, acc.at[pl.ds(off, CHUNK)])
    cp_d.wait()
    plsc.subcore_barrier()

    _gath(0).wait(); _sdesc(0).start(add=True); _gath(2).start()
    _gath(1).wait(); _sdesc(1).start(add=True)
    _sdesc(0).wait(); _gath(3).start()
    _gath(2).wait(); _sdesc(2).start(add=True)
    _sdesc(1).wait(); _gath(4).start()

    n_iter = (N_FULL - 4 - 3) // 3 + 1
    last_loop_m = 3 * (n_iter - 1) + 3 + 2

    def _body(g, carry):
        m = 3 * g + 3
        _gath_d(m, 0).wait(); _sdesc_d(m, 0).start(add=True)
        _sdesc_d(m - 1, 2).wait(); _gath_d(m + 2, 2).start()
        _gath_d(m + 1, 1).wait(); _sdesc_d(m + 1, 1).start(add=True)
        _sdesc_d(m, 0).wait(); _gath_d(m + 3, 0).start()
        _gath_d(m + 2, 2).wait(); _sdesc_d(m + 2, 2).start(add=True)
        _sdesc_d(m + 1, 1).wait(); _gath_d(m + 4, 1).start()
        return carry

    lax.fori_loop(0, n_iter, _body, 0)

    for m in range(last_loop_m + 1, N_CHUNKS):
        _gath(m).wait()
        _sdesc(m).start(add=True)
        if m + 2 < N_CHUNKS:
            _sdesc(m - 1).wait()
            _gath(m + 2).start()

    for m in range(N_CHUNKS - 3, N_CHUNKS):
        _sdesc(m).wait()

    plsc.subcore_barrier()

    @pl.when(c == 0)
    def _():
        pltpu.sync_copy(acc.at[pl.ds(s * ROWS_PER_TILE, ROWS_PER_TILE)],
                        out0_hbm.at[pl.ds(s * ROWS_PER_TILE, ROWS_PER_TILE)])

    @pl.when(c == 1)
    def _():
        pltpu.sync_copy(acc.at[pl.ds(s * ROWS_PER_TILE, ROWS_PER_TILE)],
                        out1_hbm.at[pl.ds(s * ROWS_PER_TILE, ROWS_PER_TILE)])


def _sc_aggregate(edges, features):
    mesh = plsc.VectorSubcoreMesh(core_axis_name="c", subcore_axis_name="s")
    f32 = jnp.float32
    return pl.kernel(
        _sc_agg_body,
        mesh=mesh,
        out_type=[jax.ShapeDtypeStruct((N_PAD, D), f32),
                  jax.ShapeDtypeStruct((N_PAD, D), f32)],
        scratch_types=[
            pltpu.VMEM((E_PER_TILE,), jnp.int32),
            pltpu.VMEM((E_PER_TILE,), jnp.int32),
            pltpu.VMEM((CHUNK, D), f32),
            pltpu.VMEM((CHUNK, D), f32),
            pltpu.VMEM((CHUNK, D), f32),
            pltpu.VMEM_SHARED((N_PAD, D), f32),
            pltpu.SemaphoreType.DMA,
            pltpu.SemaphoreType.DMA,
            pltpu.SemaphoreType.DMA,
            pltpu.SemaphoreType.DMA,
            pltpu.SemaphoreType.DMA,
            pltpu.SemaphoreType.DMA,
        ],
    )(edges, features)


_BM = 5000


def _mm_body(p0_ref, p1_ref, w_ref, b_ref, o_ref):
    a = p0_ref[...] + p1_ref[...]
    o_ref[...] = jnp.dot(a, w_ref[...],
                         preferred_element_type=jnp.float32) + b_ref[...]


def _linear(p0, p1, W, b2d):
    return pl.pallas_call(
        _mm_body,
        grid=(N_NODES // _BM,),
        in_specs=[
            pl.BlockSpec((_BM, D), lambda i: (i, 0)),
            pl.BlockSpec((_BM, D), lambda i: (i, 0)),
            pl.BlockSpec((D, D), lambda i: (0, 0)),
            pl.BlockSpec((1, D), lambda i: (0, 0)),
        ],
        out_specs=pl.BlockSpec((_BM, D), lambda i: (i, 0)),
        out_shape=jax.ShapeDtypeStruct((N_NODES, D), jnp.float32),
    )(p0, p1, W, b2d)


def kernel(features, edge_index, W, b):
    edges = edge_index.astype(jnp.int32).reshape(-1)
    p0, p1 = _sc_aggregate(edges, features)
    return _linear(p0, p1, W, b.reshape(1, D))


# speedup vs baseline: 1.3515x; 1.0411x over previous
import jax
import jax.numpy as jnp
from jax import lax
from jax.experimental import pallas as pl
from jax.experimental.pallas import tpu as pltpu
from jax.experimental.pallas import tpu_sc as plsc

N_NODES = 10000
N_EDGES = 320000
D = 128
NC = 2
NS = 16
E_PER_TILE = N_EDGES // (NC * NS)
CHUNK = 80
N_CHUNKS = E_PER_TILE // CHUNK
PH1 = 63
PH2 = N_CHUNKS - PH1
STG = PH1 * CHUNK
N_PAD = 10240
ROWS_PER_TILE = N_PAD // NS


def _sc_agg_body(edges_hbm, feat_hbm, out0_hbm, out1_hbm,
                 src_v, dst_v, rows_a, rows_b, rows_c, acc,
                 ga, gb, gc, sa, sb, sc):
    c = lax.axis_index("c")
    s = lax.axis_index("s")
    wid = c * NS + s
    ebase = pl.multiple_of(wid * E_PER_TILE, 8)
    dbase = pl.multiple_of(N_EDGES + wid * E_PER_TILE, 8)

    bufs = (rows_a, rows_b, rows_c)
    gsems = (ga, gb, gc)
    ssems = (sa, sb, sc)

    def _stage(base, n, buf, sem):
        return pltpu.make_async_copy(
            edges_hbm.at[pl.ds(base, n)], buf.at[pl.ds(0, n)], sem)

    def _gath(n):
        j = n % 3
        off = pl.multiple_of(n * CHUNK, 8)
        return pltpu.make_async_copy(
            feat_hbm.at[src_v.at[pl.ds(off, CHUNK)]], bufs[j], gsems[j])

    def _gath_d(m, j):
        off = pl.multiple_of(m * CHUNK, 8)
        return pltpu.make_async_copy(
            feat_hbm.at[src_v.at[pl.ds(off, CHUNK)]], bufs[j], gsems[j])

    def _sdesc(n):
        j = n % 3
        off = pl.multiple_of(n * CHUNK, 8)
        return pltpu.make_async_copy(
            bufs[j], acc.at[dst_v.at[pl.ds(off, CHUNK)]], ssems[j])

    def _sdesc_d(m, j):
        off = pl.multiple_of(m * CHUNK, 8)
        return pltpu.make_async_copy(
            bufs[j], acc.at[dst_v.at[pl.ds(off, CHUNK)]], ssems[j])

    def _phase(K):
        _gath(0).wait(); _sdesc(0).start(add=True); _gath(2).start()
        _gath(1).wait(); _sdesc(1).start(add=True)
        _sdesc(0).wait(); _gath(3).start()
        _gath(2).wait(); _sdesc(2).start(add=True)
        _sdesc(1).wait(); _gath(4).start()

        n_iter = (K - 7) // 3 + 1
        last_loop_m = 3 * (n_iter - 1) + 5

        def _body(g, carry):
            m = 3 * g + 3
            _gath_d(m, 0).wait(); _sdesc_d(m, 0).start(add=True)
            _sdesc_d(m - 1, 2).wait(); _gath_d(m + 2, 2).start()
            _gath_d(m + 1, 1).wait(); _sdesc_d(m + 1, 1).start(add=True)
            _sdesc_d(m, 0).wait(); _gath_d(m + 3, 0).start()
            _gath_d(m + 2, 2).wait(); _sdesc_d(m + 2, 2).start(add=True)
            _sdesc_d(m + 1, 1).wait(); _gath_d(m + 4, 1).start()
            return carry

        lax.fori_loop(0, n_iter, _body, 0)

        for m in range(last_loop_m + 1, K):
            _gath(m).wait()
            _sdesc(m).start(add=True)
            if m + 2 < K:
                _sdesc(m - 1).wait()
                _gath(m + 2).start()
        for m in range(K - 3, K):
            _sdesc(m).wait()

    cp_s = _stage(ebase, STG, src_v, ga)
    cp_d = _stage(dbase, STG, dst_v, sc)
    cp_s.start()
    cp_d.start()
    cp_s.wait()

    _gath(0).start()
    _gath(1).start()
    zeros16 = jnp.zeros((16,), jnp.float32)

    def _zfill(r, carry):
        for c8 in range(D // 16):
            rows_c[r, pl.ds(c8 * 16, 16)] = zeros16
        return carry

    lax.fori_loop(0, CHUNK, _zfill, 0)
    for k in range(ROWS_PER_TILE // CHUNK):
        off = pl.multiple_of(s * ROWS_PER_TILE + k * CHUNK, 8)
        pltpu.sync_copy(rows_c, acc.at[pl.ds(off, CHUNK)])
    cp_d.wait()
    plsc.subcore_barrier()

    _phase(PH1)

    cp_s2 = _stage(ebase + STG, PH2 * CHUNK, src_v, ga)
    cp_d2 = _stage(dbase + STG, PH2 * CHUNK, dst_v, sc)
    cp_s2.start()
    cp_d2.start()
    cp_s2.wait()
    _gath(0).start()
    _gath(1).start()
    cp_d2.wait()
    _phase(PH2)

    plsc.subcore_barrier()

    @pl.when(c == 0)
    def _():
        pltpu.sync_copy(acc.at[pl.ds(s * ROWS_PER_TILE, ROWS_PER_TILE)],
                        out0_hbm.at[pl.ds(s * ROWS_PER_TILE, ROWS_PER_TILE)])

    @pl.when(c == 1)
    def _():
        pltpu.sync_copy(acc.at[pl.ds(s * ROWS_PER_TILE, ROWS_PER_TILE)],
                        out1_hbm.at[pl.ds(s * ROWS_PER_TILE, ROWS_PER_TILE)])


def _sc_aggregate(edges, features):
    mesh = plsc.VectorSubcoreMesh(core_axis_name="c", subcore_axis_name="s")
    f32 = jnp.float32
    return pl.kernel(
        _sc_agg_body,
        mesh=mesh,
        out_type=[jax.ShapeDtypeStruct((N_PAD, D), f32),
                  jax.ShapeDtypeStruct((N_PAD, D), f32)],
        scratch_types=[
            pltpu.VMEM((STG,), jnp.int32),
            pltpu.VMEM((STG,), jnp.int32),
            pltpu.VMEM((CHUNK, D), f32),
            pltpu.VMEM((CHUNK, D), f32),
            pltpu.VMEM((CHUNK, D), f32),
            pltpu.VMEM_SHARED((N_PAD, D), f32),
            pltpu.SemaphoreType.DMA,
            pltpu.SemaphoreType.DMA,
            pltpu.SemaphoreType.DMA,
            pltpu.SemaphoreType.DMA,
            pltpu.SemaphoreType.DMA,
            pltpu.SemaphoreType.DMA,
        ],
    )(edges, features)


_BM = 5000


def _mm_body(p0_ref, p1_ref, w_ref, b_ref, o_ref):
    a = p0_ref[...] + p1_ref[...]
    o_ref[...] = jnp.dot(a, w_ref[...],
                         preferred_element_type=jnp.float32) + b_ref[...]


def _linear(p0, p1, W, b2d):
    return pl.pallas_call(
        _mm_body,
        grid=(N_NODES // _BM,),
        in_specs=[
            pl.BlockSpec((_BM, D), lambda i: (i, 0)),
            pl.BlockSpec((_BM, D), lambda i: (i, 0)),
            pl.BlockSpec((D, D), lambda i: (0, 0)),
            pl.BlockSpec((1, D), lambda i: (0, 0)),
        ],
        out_specs=pl.BlockSpec((_BM, D), lambda i: (i, 0)),
        out_shape=jax.ShapeDtypeStruct((N_NODES, D), jnp.float32),
    )(p0, p1, W, b2d)


def kernel(features, edge_index, W, b):
    edges = edge_index.astype(jnp.int32).reshape(-1)
    p0, p1 = _sc_aggregate(edges, features)
    return _linear(p0, p1, W, b.reshape(1, D))
